# baseline (speedup 1.0000x reference)
import functools

import jax
import jax.numpy as jnp
from jax import lax
from jax.experimental import pallas as pl
from jax.experimental.pallas import tpu as pltpu
from jax.experimental.pallas import tpu_sc as plsc

K = 5
K3 = 125
_NNODES = [10000, 1250, 640, 320, 160, 80, 40]

_NC = 2
_NS = 16
_SLAB = 128


def _split8(n, parts):
    assert n % 8 == 0
    q, r = divmod(n // 8, parts)
    out, st = [], 0
    for i in range(parts):
        cnt = (q + (1 if i < r else 0)) * 8
        out.append((st, cnt))
        st += cnt
    return out


def _sc_gather(x, src_pad, e_pad, fin):
    n_words = x.shape[0] * fin
    nt = _NC * _NS
    rows_per_tile = e_pad // nt
    nslab = rows_per_tile // _SLAB
    nvec = _SLAB * fin // 16
    mesh = plsc.VectorSubcoreMesh(core_axis_name="c", subcore_axis_name="s")

    @functools.partial(
        pl.kernel,
        out_type=jax.ShapeDtypeStruct((e_pad * fin,), jnp.float32),
        mesh=mesh,
        compiler_params=pltpu.CompilerParams(needs_layout_passes=False),
        scratch_types=[
            pltpu.VMEM((n_words,), jnp.float32),
            pltpu.VMEM((_SLAB,), jnp.int32),
            pltpu.VMEM((_SLAB * fin,), jnp.float32),
        ],
    )
    def gather_k(x_hbm, src_hbm, out_hbm, x_v, idx_v, row_v):
        c = lax.axis_index("c")
        s = lax.axis_index("s")
        wid = s * _NC + c
        base = wid * rows_per_tile
        pltpu.sync_copy(x_hbm, x_v)

        def body(j, carry):
            off = pl.multiple_of(base + j * _SLAB, _SLAB)
            pltpu.sync_copy(src_hbm.at[pl.ds(off, _SLAB)], idx_v)

            lgf = fin.bit_length() - 1

            def vec(v, carry2):
                flat = v * 16 + lax.iota(jnp.int32, 16)
                r = lax.shift_right_logical(flat, lgf)
                col = flat & (fin - 1)
                srcv = plsc.load_gather(idx_v, [r])
                xv = plsc.load_gather(x_v, [lax.shift_left(srcv, lgf) + col])
                row_v[pl.ds(pl.multiple_of(v * 16, 16), 16)] = xv
                return carry2

            lax.fori_loop(0, nvec, vec, 0)
            pltpu.sync_copy(
                row_v, out_hbm.at[pl.ds(pl.multiple_of(off * fin, _SLAB), _SLAB * fin)])
            return carry

        lax.fori_loop(0, nslab, body, 0)

    return gather_k(x.reshape(n_words), src_pad).reshape(e_pad, fin)


_SPMEM_WORDS = 1_700_000


def _scatter_chunks(rows, d):
    npass = 1
    while (-(-rows // (2 * npass)) + 8) * d > _SPMEM_WORDS:
        npass += 1
    nch = 2 * npass
    if rows % 8 == 0:
        q, r = divmod(rows // 8, nch)
        sizes = [(q + (1 if i < r else 0)) * 8 for i in range(nch)]
        return sizes, False
    assert nch == 2
    return [-(-(rows // 2) // 8) * 8] * 2, True


def _sc_scatter(msgs, seg, zeros, num_nodes, d):
    rows = num_nodes * K3
    half = rows // 2
    sizes, padded = _scatter_chunks(rows, d)
    starts = [sum(sizes[:i]) for i in range(len(sizes))]
    npass = len(sizes) // 2
    max_chunk = max(sizes)
    sp_rows = max_chunk + 8
    junk = max_chunk
    out_rows = sum(sizes)
    r8 = msgs.shape[0]
    rows_per_tile = r8 // _NS
    nslab = rows_per_tile // _SLAB
    mesh = plsc.VectorSubcoreMesh(core_axis_name="c", subcore_axis_name="s")

    @functools.partial(
        pl.kernel,
        out_type=jax.ShapeDtypeStruct((out_rows, d), jnp.float32),
        mesh=mesh,
        compiler_params=pltpu.CompilerParams(
            needs_layout_passes=False, use_tc_tiling_on_sc=False),
        scratch_types=[
            pltpu.VMEM((_SLAB,), jnp.int32),
            pltpu.VMEM((_SLAB,), jnp.int32),
            pltpu.VMEM((_SLAB, d), jnp.float32),
            pltpu.VMEM_SHARED((sp_rows, d), jnp.float32),
        ],
    )
    def scatter_k(m_hbm, seg_hbm, z_hbm, out_hbm, segv, lidxv, msgv, spmem):
        c = lax.axis_index("c")
        s = lax.axis_index("s")
        mbase = s * rows_per_tile

        for p in range(npass):
            lo = jnp.where(c == 0, starts[2 * p], starts[2 * p + 1])
            span = jnp.where(
                c == 0,
                min(sizes[2 * p], rows - starts[2 * p]),
                min(sizes[2 * p + 1], rows - starts[2 * p + 1]))

            for i, (st, cnt) in enumerate(_split8(sp_rows, _NS)):
                @pl.when(s == i)
                def _zero(st=st, cnt=cnt):
                    pltpu.sync_copy(z_hbm.at[pl.ds(st, cnt), :],
                                    spmem.at[pl.ds(st, cnt), :])
            plsc.subcore_barrier()

            def body(j, carry):
                off = pl.multiple_of(mbase + j * _SLAB, _SLAB)
                pltpu.sync_copy(seg_hbm.at[pl.ds(off, _SLAB)], segv)

                def remap(i, carry2):
                    sl = pl.ds(pl.multiple_of(i * 16, 16), 16)
                    t = segv[sl] - lo
                    ok = (t >= 0) & (t < span)
                    lidxv[sl] = jnp.where(ok, t, junk)
                    return carry2

                lax.fori_loop(0, _SLAB // 16, remap, 0)
                pltpu.sync_copy(m_hbm.at[pl.ds(off, _SLAB), :], msgv)
                pltpu.sync_copy(msgv, spmem.at[lidxv], add=True)
                return carry

            lax.fori_loop(0, nslab, body, 0)
            plsc.subcore_barrier()

            for cc in range(2):
                q = 2 * p + cc
                for i, (st, cnt) in enumerate(_split8(sizes[q], _NS)):
                    @pl.when((c == cc) & (s == i))
                    def _out(st=st, cnt=cnt, q=q):
                        pltpu.sync_copy(
                            spmem.at[pl.ds(st, cnt), :],
                            out_hbm.at[pl.ds(starts[q] + st, cnt), :])
            if p + 1 < npass:
                plsc.subcore_barrier()

    a_pad = scatter_k(msgs, seg, zeros)
    return a_pad if out_rows == rows else a_pad[:rows]


def _edge_prep(edge_attr, dst, x_src, e_pad, num_nodes):
    e = edge_attr.shape[0]
    fin = x_src.shape[1]
    junk = num_nodes * K3
    p = jnp.clip(edge_attr, 0.0, 1.0) * (K - 1)
    bot = jnp.clip(jnp.floor(p), 0.0, float(K - 2))
    frac = p - bot
    boti = bot.astype(jnp.int32)
    bs, segs = [], []
    for c0 in range(2):
        for c1 in range(2):
            for c2 in range(2):
                b0 = frac[:, 0] if c0 else 1.0 - frac[:, 0]
                b1 = frac[:, 1] if c1 else 1.0 - frac[:, 1]
                b2 = frac[:, 2] if c2 else 1.0 - frac[:, 2]
                wi = (boti[:, 0] + c0) + (boti[:, 1] + c1) * K + (boti[:, 2] + c2) * (K * K)
                bs.append(b0 * b1 * b2)
                segs.append(dst * K3 + wi)
    b8 = jnp.stack(bs, axis=1)
    seg8 = jnp.stack(segs, axis=1)
    b8 = jnp.pad(b8, ((0, e_pad - e), (0, 0)))
    seg8 = jnp.pad(seg8, ((0, e_pad - e), (0, 0)), constant_values=junk)
    feat = b8[:, :, None] * x_src[:, None, :]
    if fin < 128:
        d = -(-(fin + 1) // 8) * 8
        pad = jnp.zeros((e_pad, 8, d - fin - 1), jnp.float32)
        msgs = jnp.concatenate([feat, b8[:, :, None], pad], axis=2)
    else:
        msgs = feat
        d = fin
    return msgs.reshape(8 * e_pad, d), seg8.reshape(8 * e_pad)


def _dense_tail_body(a_ref, wext_ref, x_ref, r_ref, bias_ref, p_ref, acc_ref,
                     *, fout, last_level, dst_ref=None):
    i = pl.program_id(0)

    @pl.when(i == 0)
    def _init():
        acc_ref[...] = jnp.zeros_like(acc_ref)

    z = jnp.dot(a_ref[...], wext_ref[...], preferred_element_type=jnp.float32)
    if dst_ref is None:
        conv = z[:, :fout]
        deg = z[:, fout:fout + 1]
    else:
        conv = z
        n_blk = a_ref.shape[0]
        nodes = jax.lax.broadcasted_iota(jnp.int32, (n_blk, 1), 0)
        onehot = (nodes == dst_ref[0]).astype(jnp.float32)
        deg = jnp.dot(onehot, jnp.ones((onehot.shape[1], 1), jnp.float32),
                      preferred_element_type=jnp.float32)
    h = conv / jnp.maximum(deg, 1.0)
    h = h + jnp.dot(x_ref[...], r_ref[...], preferred_element_type=jnp.float32)
    h = h + bias_ref[...]
    h = jnp.where(h > 0, h, jnp.exp(jnp.minimum(h, 0.0)) - 1.0)
    pooled = jnp.dot(p_ref[...].T, h, preferred_element_type=jnp.float32)
    if last_level:
        acc_ref[...] = jnp.max(pooled, axis=0, keepdims=True)
    else:
        acc_ref[...] += pooled


def _dense_tail(A, W, x, R, bias, P, num_nodes, n_next, block_n, last_level,
                dst=None):
    fin = x.shape[1]
    fout = R.shape[1]
    if dst is None:
        d = -(-(fin + 1) // 8) * 8
        wcols = K3 * d
        W_ext = jnp.concatenate([W, jnp.zeros((K3, fin, 1), W.dtype)], axis=2)
        brow = jnp.zeros((K3, 1, fout + 1), W.dtype).at[:, 0, fout].set(1.0)
        zrow = jnp.zeros((K3, d - fin - 1, fout + 1), W.dtype)
        W_ext = jnp.concatenate([W_ext, brow, zrow], axis=1).reshape(
            wcols, fout + 1)
        zcols = fout + 1
    else:
        wcols = K3 * fin
        W_ext = W.reshape(wcols, fout)
        zcols = fout

    A2 = A.reshape(num_nodes, wcols)
    grid = (num_nodes // block_n,)
    out_rows = 1 if last_level else n_next
    in_specs = [
        pl.BlockSpec((block_n, wcols), lambda i: (i, 0)),
        pl.BlockSpec((wcols, zcols), lambda i: (0, 0)),
        pl.BlockSpec((block_n, fin), lambda i: (i, 0)),
        pl.BlockSpec((fin, fout), lambda i: (0, 0)),
        pl.BlockSpec((1, fout), lambda i: (0, 0)),
        pl.BlockSpec((block_n, n_next), lambda i: (i, 0)),
    ]
    args = [A2, W_ext, x, R, bias.reshape(1, fout), P]
    if dst is not None:
        e2 = -(-dst.shape[0] // 8) * 8
        dst2 = jnp.pad(dst, (0, e2 - dst.shape[0]),
                       constant_values=-1).reshape(1, e2)
        in_specs.append(pl.BlockSpec((1, e2), lambda i: (0, 0)))
        args.append(dst2)

    def body(*refs):
        if dst is not None:
            a, w, xr, r, b, p, dref, acc = refs
            _dense_tail_body(a, w, xr, r, b, p, acc, fout=fout,
                             last_level=last_level, dst_ref=dref)
        else:
            a, w, xr, r, b, p, acc = refs
            _dense_tail_body(a, w, xr, r, b, p, acc, fout=fout,
                             last_level=last_level)

    return pl.pallas_call(
        body,
        grid=grid,
        in_specs=in_specs,
        out_specs=pl.BlockSpec((out_rows, fout), lambda i: (0, 0)),
        out_shape=jax.ShapeDtypeStruct((out_rows, fout), jnp.float32),
    )(*args)


def kernel(x, edge_index0, edge_index1, edge_index2, edge_index3, edge_index4, edge_index5, edge_attr0, edge_attr1, edge_attr2, edge_attr3, edge_attr4, edge_attr5, P01, P12, P23, P34, P45, P56, W1, W2, W3, W4, W5, W6, R1, R2, R3, R4, R5, R6, B1, B2, B3, B4, B5, B6):
    EI = [edge_index0, edge_index1, edge_index2, edge_index3, edge_index4, edge_index5]
    EA = [edge_attr0, edge_attr1, edge_attr2, edge_attr3, edge_attr4, edge_attr5]
    Ps = [P01, P12, P23, P34, P45, P56]
    Ws = [W1, W2, W3, W4, W5, W6]
    Rs = [R1, R2, R3, R4, R5, R6]
    Bs = [B1, B2, B3, B4, B5, B6]
    BLOCK = [1000, 1250, 640, 320, 160, 80]

    h = x
    for l in range(6):
        n = _NNODES[l]
        e = EI[l].shape[1]
        e_pad = -(-e // 4096) * 4096
        fin = h.shape[1]
        d = -(-(fin + 1) // 8) * 8 if fin < 128 else fin
        src = jnp.pad(EI[l][0], (0, e_pad - e))
        x_src = _sc_gather(h, src, e_pad, fin)
        msgs, seg = _edge_prep(EA[l], EI[l][1], x_src, e_pad, n)
        sizes_l, _ = _scatter_chunks(n * K3, d)
        zeros = jnp.zeros((max(sizes_l) + 8, d), jnp.float32)
        A = _sc_scatter(msgs, seg, zeros, n, d)
        h = _dense_tail(A, Ws[l], h, Rs[l], Bs[l], Ps[l],
                        n, _NNODES[l + 1], BLOCK[l], last_level=(l == 5),
                        dst=None if fin < 128 else EI[l][1])
    return h


# speedup vs baseline: 1.0104x; 1.0104x over previous
import functools

import jax
import jax.numpy as jnp
from jax import lax
from jax.experimental import pallas as pl
from jax.experimental.pallas import tpu as pltpu
from jax.experimental.pallas import tpu_sc as plsc

K = 5
K3 = 125
_NNODES = [10000, 1250, 640, 320, 160, 80, 40]

_NC = 2
_NS = 16
_SLAB = 128


def _split8(n, parts):
    assert n % 8 == 0
    q, r = divmod(n // 8, parts)
    out, st = [], 0
    for i in range(parts):
        cnt = (q + (1 if i < r else 0)) * 8
        out.append((st, cnt))
        st += cnt
    return out


def _sc_gather(x, src_pad, e_pad, fin):
    n_words = x.shape[0] * fin
    nt = _NC * _NS
    rows_per_tile = e_pad // nt
    nslab = rows_per_tile // _SLAB
    nvec = _SLAB * fin // 16
    mesh = plsc.VectorSubcoreMesh(core_axis_name="c", subcore_axis_name="s")

    @functools.partial(
        pl.kernel,
        out_type=jax.ShapeDtypeStruct((e_pad * fin,), jnp.float32),
        mesh=mesh,
        compiler_params=pltpu.CompilerParams(needs_layout_passes=False),
        scratch_types=[
            pltpu.VMEM((n_words,), jnp.float32),
            pltpu.VMEM((_SLAB,), jnp.int32),
            pltpu.VMEM((_SLAB * fin,), jnp.float32),
        ],
    )
    def gather_k(x_hbm, src_hbm, out_hbm, x_v, idx_v, row_v):
        c = lax.axis_index("c")
        s = lax.axis_index("s")
        wid = s * _NC + c
        base = wid * rows_per_tile
        pltpu.sync_copy(x_hbm, x_v)

        def body(j, carry):
            off = pl.multiple_of(base + j * _SLAB, _SLAB)
            pltpu.sync_copy(src_hbm.at[pl.ds(off, _SLAB)], idx_v)

            lgf = fin.bit_length() - 1

            def vec(v, carry2):
                flat = v * 16 + lax.iota(jnp.int32, 16)
                r = lax.shift_right_logical(flat, lgf)
                col = flat & (fin - 1)
                srcv = plsc.load_gather(idx_v, [r])
                xv = plsc.load_gather(x_v, [lax.shift_left(srcv, lgf) + col])
                row_v[pl.ds(pl.multiple_of(v * 16, 16), 16)] = xv
                return carry2

            lax.fori_loop(0, nvec, vec, 0)
            pltpu.sync_copy(
                row_v, out_hbm.at[pl.ds(pl.multiple_of(off * fin, _SLAB), _SLAB * fin)])
            return carry

        lax.fori_loop(0, nslab, body, 0)

    return gather_k(x.reshape(n_words), src_pad).reshape(e_pad, fin)


_SPMEM_WORDS = 1_700_000


def _dpad(fin):
    if fin >= 128:
        return fin
    return -(-(fin + 1) // 8) * 8


def _scatter_chunks(rows, d):
    npass = 1
    while (-(-rows // (2 * npass)) + 8) * d > _SPMEM_WORDS:
        npass += 1
    nch = 2 * npass
    if rows % 8 == 0:
        q, r = divmod(rows // 8, nch)
        sizes = [(q + (1 if i < r else 0)) * 8 for i in range(nch)]
        return sizes, False
    assert nch == 2
    return [-(-(rows // 2) // 8) * 8] * 2, True


def _sc_scatter(msgs, seg, zeros, num_nodes, d):
    rows = num_nodes * K3
    half = rows // 2
    sizes, padded = _scatter_chunks(rows, d)
    starts = [sum(sizes[:i]) for i in range(len(sizes))]
    npass = len(sizes) // 2
    max_chunk = max(sizes)
    sp_rows = max_chunk + 8
    junk = max_chunk
    out_rows = sum(sizes)
    r8 = msgs.shape[0]
    rows_per_tile = r8 // _NS
    nslab = rows_per_tile // _SLAB
    kb = 8 if d < 128 else 4
    nbatch = nslab // kb
    mesh = plsc.VectorSubcoreMesh(core_axis_name="c", subcore_axis_name="s")

    @functools.partial(
        pl.kernel,
        out_type=jax.ShapeDtypeStruct((out_rows, d), jnp.float32),
        mesh=mesh,
        compiler_params=pltpu.CompilerParams(
            needs_layout_passes=False, use_tc_tiling_on_sc=False),
        scratch_types=[
            pltpu.VMEM((kb * _SLAB,), jnp.int32),
            pltpu.VMEM((kb * _SLAB, d), jnp.float32),
            [pltpu.VMEM((_SLAB,), jnp.int32) for _ in range(kb)],
            pltpu.VMEM_SHARED((sp_rows, d), jnp.float32),
            pltpu.SemaphoreType.DMA,
        ],
    )
    def scatter_k(m_hbm, seg_hbm, z_hbm, out_hbm, segv, msgv, lidxb, spmem,
                  sem):
        c = lax.axis_index("c")
        s = lax.axis_index("s")
        mbase = s * rows_per_tile

        for p in range(npass):
            lo = jnp.where(c == 0, starts[2 * p], starts[2 * p + 1])
            span = jnp.where(
                c == 0,
                min(sizes[2 * p], rows - starts[2 * p]),
                min(sizes[2 * p + 1], rows - starts[2 * p + 1]))

            for i, (st, cnt) in enumerate(_split8(sp_rows, _NS)):
                @pl.when(s == i)
                def _zero(st=st, cnt=cnt):
                    pltpu.sync_copy(z_hbm.at[pl.ds(st, cnt), :],
                                    spmem.at[pl.ds(st, cnt), :])
            plsc.subcore_barrier()

            def body(g, carry):
                off = pl.multiple_of(mbase + g * (kb * _SLAB), kb * _SLAB)
                pltpu.sync_copy(seg_hbm.at[pl.ds(off, kb * _SLAB)], segv)
                pltpu.sync_copy(m_hbm.at[pl.ds(off, kb * _SLAB), :], msgv)
                for k in range(kb):

                    def remap(i, carry2, k=k):
                        sl = pl.ds(pl.multiple_of(k * _SLAB + i * 16, 16), 16)
                        t = segv[sl] - lo
                        ok = (t >= 0) & (t < span)
                        lidxb[k][pl.ds(pl.multiple_of(i * 16, 16), 16)] = (
                            jnp.where(ok, t, junk))
                        return carry2

                    lax.fori_loop(0, _SLAB // 16, remap, 0)
                handles = [
                    pltpu.async_copy(
                        msgv.at[pl.ds(k * _SLAB, _SLAB), :],
                        spmem.at[lidxb[k]], sem, add=True)
                    for k in range(kb)
                ]
                for h in handles:
                    h.wait()
                return carry

            lax.fori_loop(0, nbatch, body, 0)
            plsc.subcore_barrier()

            for cc in range(2):
                q = 2 * p + cc
                for i, (st, cnt) in enumerate(_split8(sizes[q], _NS)):
                    @pl.when((c == cc) & (s == i))
                    def _out(st=st, cnt=cnt, q=q):
                        pltpu.sync_copy(
                            spmem.at[pl.ds(st, cnt), :],
                            out_hbm.at[pl.ds(starts[q] + st, cnt), :])
            if p + 1 < npass:
                plsc.subcore_barrier()

    a_pad = scatter_k(msgs, seg, zeros)
    return a_pad if out_rows == rows else a_pad[:rows]


def _edge_prep(edge_attr, dst, x_src, e_pad, num_nodes):
    e = edge_attr.shape[0]
    fin = x_src.shape[1]
    junk = num_nodes * K3
    p = jnp.clip(edge_attr, 0.0, 1.0) * (K - 1)
    bot = jnp.clip(jnp.floor(p), 0.0, float(K - 2))
    frac = p - bot
    boti = bot.astype(jnp.int32)
    bs, segs = [], []
    for c0 in range(2):
        for c1 in range(2):
            for c2 in range(2):
                b0 = frac[:, 0] if c0 else 1.0 - frac[:, 0]
                b1 = frac[:, 1] if c1 else 1.0 - frac[:, 1]
                b2 = frac[:, 2] if c2 else 1.0 - frac[:, 2]
                wi = (boti[:, 0] + c0) + (boti[:, 1] + c1) * K + (boti[:, 2] + c2) * (K * K)
                bs.append(b0 * b1 * b2)
                segs.append(dst * K3 + wi)
    b8 = jnp.stack(bs, axis=1)
    seg8 = jnp.stack(segs, axis=1)
    b8 = jnp.pad(b8, ((0, e_pad - e), (0, 0)))
    seg8 = jnp.pad(seg8, ((0, e_pad - e), (0, 0)), constant_values=junk)
    feat = b8[:, :, None] * x_src[:, None, :]
    if fin < 128:
        d = _dpad(fin)
        pad = jnp.zeros((e_pad, 8, d - fin - 1), jnp.float32)
        msgs = jnp.concatenate([feat, b8[:, :, None], pad], axis=2)
    else:
        msgs = feat
        d = fin
    return msgs.reshape(8 * e_pad, d), seg8.reshape(8 * e_pad)


def _dense_tail_body(a_ref, wext_ref, x_ref, r_ref, bias_ref, p_ref, acc_ref,
                     *, fout, last_level, dst_ref=None):
    i = pl.program_id(0)

    @pl.when(i == 0)
    def _init():
        acc_ref[...] = jnp.zeros_like(acc_ref)

    z = jnp.dot(a_ref[...], wext_ref[...], preferred_element_type=jnp.float32)
    if dst_ref is None:
        conv = z[:, :fout]
        deg = z[:, fout:fout + 1]
    else:
        conv = z
        n_blk = a_ref.shape[0]
        nodes = jax.lax.broadcasted_iota(jnp.int32, (n_blk, 1), 0)
        onehot = (nodes == dst_ref[0]).astype(jnp.float32)
        deg = jnp.dot(onehot, jnp.ones((onehot.shape[1], 1), jnp.float32),
                      preferred_element_type=jnp.float32)
    h = conv / jnp.maximum(deg, 1.0)
    h = h + jnp.dot(x_ref[...], r_ref[...], preferred_element_type=jnp.float32)
    h = h + bias_ref[...]
    h = jnp.where(h > 0, h, jnp.exp(jnp.minimum(h, 0.0)) - 1.0)
    pooled = jnp.dot(p_ref[...].T, h, preferred_element_type=jnp.float32)
    if last_level:
        acc_ref[...] = jnp.max(pooled, axis=0, keepdims=True)
    else:
        acc_ref[...] += pooled


def _dense_tail(A, W, x, R, bias, P, num_nodes, n_next, block_n, last_level,
                dst=None):
    fin = x.shape[1]
    fout = R.shape[1]
    if dst is None:
        d = _dpad(fin)
        wcols = K3 * d
        W_ext = jnp.concatenate([W, jnp.zeros((K3, fin, 1), W.dtype)], axis=2)
        brow = jnp.zeros((K3, 1, fout + 1), W.dtype).at[:, 0, fout].set(1.0)
        zrow = jnp.zeros((K3, d - fin - 1, fout + 1), W.dtype)
        W_ext = jnp.concatenate([W_ext, brow, zrow], axis=1).reshape(
            wcols, fout + 1)
        zcols = fout + 1
    else:
        wcols = K3 * fin
        W_ext = W.reshape(wcols, fout)
        zcols = fout

    A2 = A.reshape(num_nodes, wcols)
    grid = (num_nodes // block_n,)
    out_rows = 1 if last_level else n_next
    in_specs = [
        pl.BlockSpec((block_n, wcols), lambda i: (i, 0)),
        pl.BlockSpec((wcols, zcols), lambda i: (0, 0)),
        pl.BlockSpec((block_n, fin), lambda i: (i, 0)),
        pl.BlockSpec((fin, fout), lambda i: (0, 0)),
        pl.BlockSpec((1, fout), lambda i: (0, 0)),
        pl.BlockSpec((block_n, n_next), lambda i: (i, 0)),
    ]
    args = [A2, W_ext, x, R, bias.reshape(1, fout), P]
    if dst is not None:
        e2 = -(-dst.shape[0] // 8) * 8
        dst2 = jnp.pad(dst, (0, e2 - dst.shape[0]),
                       constant_values=-1).reshape(1, e2)
        in_specs.append(pl.BlockSpec((1, e2), lambda i: (0, 0)))
        args.append(dst2)

    def body(*refs):
        if dst is not None:
            a, w, xr, r, b, p, dref, acc = refs
            _dense_tail_body(a, w, xr, r, b, p, acc, fout=fout,
                             last_level=last_level, dst_ref=dref)
        else:
            a, w, xr, r, b, p, acc = refs
            _dense_tail_body(a, w, xr, r, b, p, acc, fout=fout,
                             last_level=last_level)

    return pl.pallas_call(
        body,
        grid=grid,
        in_specs=in_specs,
        out_specs=pl.BlockSpec((out_rows, fout), lambda i: (0, 0)),
        out_shape=jax.ShapeDtypeStruct((out_rows, fout), jnp.float32),
    )(*args)


def kernel(x, edge_index0, edge_index1, edge_index2, edge_index3, edge_index4, edge_index5, edge_attr0, edge_attr1, edge_attr2, edge_attr3, edge_attr4, edge_attr5, P01, P12, P23, P34, P45, P56, W1, W2, W3, W4, W5, W6, R1, R2, R3, R4, R5, R6, B1, B2, B3, B4, B5, B6):
    EI = [edge_index0, edge_index1, edge_index2, edge_index3, edge_index4, edge_index5]
    EA = [edge_attr0, edge_attr1, edge_attr2, edge_attr3, edge_attr4, edge_attr5]
    Ps = [P01, P12, P23, P34, P45, P56]
    Ws = [W1, W2, W3, W4, W5, W6]
    Rs = [R1, R2, R3, R4, R5, R6]
    Bs = [B1, B2, B3, B4, B5, B6]
    BLOCK = [1000, 1250, 640, 320, 160, 80]

    h = x
    for l in range(6):
        n = _NNODES[l]
        e = EI[l].shape[1]
        e_pad = -(-e // 4096) * 4096
        fin = h.shape[1]
        d = _dpad(fin)
        src = jnp.pad(EI[l][0], (0, e_pad - e))
        x_src = _sc_gather(h, src, e_pad, fin)
        msgs, seg = _edge_prep(EA[l], EI[l][1], x_src, e_pad, n)
        sizes_l, _ = _scatter_chunks(n * K3, d)
        zeros = jnp.zeros((max(sizes_l) + 8, d), jnp.float32)
        A = _sc_scatter(msgs, seg, zeros, n, d)
        h = _dense_tail(A, Ws[l], h, Rs[l], Bs[l], Ps[l],
                        n, _NNODES[l + 1], BLOCK[l], last_level=(l == 5),
                        dst=None if fin < 128 else EI[l][1])
    return h
